# baseline (speedup 1.0000x reference)
import functools
import math

import jax
import jax.numpy as jnp
from jax import lax
from jax.experimental import pallas as pl
from jax.experimental.pallas import tpu as pltpu
from jax.experimental.pallas import tpu_sc as plsc

B = 4096
HIST = 50
EDIM = 64
N_UID = 100000
N_CAT = 1000
NCATP = 1024
NC, NS = 2, 16
NW = NC * NS
PW = B // NW
CB = 8
NCH = PW // CB
GCH = 80
NG = (CB * HIST) // GCH
_UNR = 5

_f32 = jnp.float32
_i32 = jnp.int32


def _hist_pass(tab, idxv, hb0, hb1, sem0, sem1, sum_all):
    def fire(c, buf, sem):
        for g in range(NG):
            src = tab.at[idxv.at[pl.ds(c * CB * HIST + g * GCH, GCH)]]
            pltpu.async_copy(src, buf.at[pl.ds(g * GCH, GCH)], sem)

    def drain(buf, sem):
        pltpu.make_async_copy(tab.at[pl.ds(0, CB * HIST)], buf, sem).wait()

    def reduce(c, buf):
        for b in range(CB):
            def body(j, acc):
                for u in range(_UNR):
                    r = b * HIST + j * _UNR + u
                    acc = tuple(acc[d] + buf[r, pl.ds(16 * d, 16)]
                                for d in range(4))
                return acc
            z = jnp.zeros((16,), _f32)
            a = lax.fori_loop(0, HIST // _UNR, body, (z, z, z, z))
            row = c * CB + b
            for d in range(4):
                sum_all[row, pl.ds(16 * d, 16)] = a[d]

    fire(0, hb0, sem0)

    def pair(i, carry):
        c0 = 2 * i
        c1 = c0 + 1
        fire(c1, hb1, sem1)
        drain(hb0, sem0)
        reduce(c0, hb0)

        @pl.when(c1 + 1 < NCH)
        def _():
            fire(c1 + 1, hb0, sem0)

        drain(hb1, sem1)
        reduce(c1, hb1)
        return carry

    lax.fori_loop(0, NCH // 2, pair, 0)


def _sc_hist_body(idx_i, hidx_i, tab,
                  emb_o, sum_o,
                  idxv, hidxv, rows, hb0, hb1, ssum,
                  isem, hsem, usem, gsem0, gsem1, osem):
    wid = lax.axis_index("s") * NC + lax.axis_index("c")
    base = wid * PW
    b50 = base * HIST

    d_h = pltpu.async_copy(hidx_i.at[pl.ds(b50, PW * HIST)], hidxv, hsem)
    d_i = pltpu.async_copy(idx_i.at[pl.ds(base, PW)], idxv, isem)
    d_i.wait()
    g = pltpu.async_copy(tab.at[idxv], rows, usem)
    d_h.wait()
    _hist_pass(tab, hidxv, hb0, hb1, gsem0, gsem1, ssum)
    g.wait()
    dst = pl.ds(base, PW)
    o1 = pltpu.async_copy(rows, emb_o.at[dst], osem)
    o2 = pltpu.async_copy(ssum, sum_o.at[dst], osem)
    o1.wait()
    o2.wait()


def _flat_gather(idxv, nrow, flat_t, idxb, rowsf, gsem):
    foffs = [(lax.iota(_i32, 16) + 16 * k) * nrow for k in range(4)]

    def build(c, carry):
        for b in range(CB):
            r = c * CB + b
            bc = plsc.load_gather(idxv, [jnp.full((16,), r, _i32)])
            for k in range(4):
                idxb[pl.ds(r * EDIM + 16 * k, 16)] = bc + foffs[k]
        return carry

    lax.fori_loop(0, NCH, build, 0)

    def fire(d, carry):
        src = flat_t.at[idxb.at[pl.ds(d * 128, 128)]]
        pltpu.async_copy(src, rowsf.at[pl.ds(d * 128, 128)], gsem)
        return carry

    lax.fori_loop(0, PW * EDIM // 128, fire, 0)
    pltpu.make_async_copy(flat_t.at[pl.ds(0, PW * EDIM)], rowsf, gsem).wait()


def _sc_flat_body(uid_i, cat_i, ut, ct,
                  ue_o, ce_o,
                  uidv, catv, idxb, rowsf,
                  iusem, icsem, gsem, osem):
    wid = lax.axis_index("s") * NC + lax.axis_index("c")
    base = wid * PW
    d_u = pltpu.async_copy(uid_i.at[pl.ds(base, PW)], uidv, iusem)
    d_c = pltpu.async_copy(cat_i.at[pl.ds(base, PW)], catv, icsem)
    dst = pl.ds(base * EDIM, PW * EDIM)
    d_u.wait()
    _flat_gather(uidv, N_UID, ut, idxb, rowsf, gsem)
    pltpu.async_copy(rowsf, ue_o.at[dst], osem).wait()
    d_c.wait()
    _flat_gather(catv, N_CAT, ct, idxb, rowsf, gsem)
    pltpu.async_copy(rowsf, ce_o.at[dst], osem).wait()


def _sc_cnt_body(hidx_i, cnt_o, hidxv, cb0, cb1, hsem, osem):
    wid = lax.axis_index("s") * NC + lax.axis_index("c")
    base = wid * PW
    pltpu.async_copy(hidx_i.at[pl.ds(base * HIST, PW * HIST)], hidxv,
                     hsem).wait()
    zero = jnp.zeros((16,), _f32)
    one = jnp.ones((16,), _f32)
    iota = lax.iota(_i32, 16)

    def chunk(c, cbuf):
        def zbody(z, carry):
            cbuf[pl.ds(z * 16, 16)] = zero
            return carry
        lax.fori_loop(0, CB * NCATP // 16, zbody, 0)
        for g in range(CB * HIST // 16):
            rowv = (g * 16 + iota) // HIST
            colv = hidxv[pl.ds(c * CB * HIST + g * 16, 16)]
            plsc.addupdate_scatter(cbuf, [rowv * NCATP + colv], one)
        pltpu.async_copy(cbuf,
                         cnt_o.at[pl.ds((base + c * CB) * NCATP, CB * NCATP)],
                         osem)

    def pair(i, carry):
        chunk(2 * i, cb0)
        chunk(2 * i + 1, cb1)
        pltpu.make_async_copy(cnt_o.at[pl.ds(0, CB * NCATP)], cb0,
                              osem).wait()
        pltpu.make_async_copy(cnt_o.at[pl.ds(0, CB * NCATP)], cb1,
                              osem).wait()
        return carry

    lax.fori_loop(0, NCH // 2, pair, 0)


_SC_PARAMS = dict(
    mesh=plsc.VectorSubcoreMesh(core_axis_name="c", subcore_axis_name="s"),
    compiler_params=pltpu.CompilerParams(use_tc_tiling_on_sc=False,
                                         needs_layout_passes=False),
)

_sc_hist = functools.partial(
    pl.kernel,
    out_type=[jax.ShapeDtypeStruct((B, EDIM), _f32),
              jax.ShapeDtypeStruct((B, EDIM), _f32)],
    scratch_types=[
        pltpu.VMEM((PW,), _i32),
        pltpu.VMEM((PW * HIST,), _i32),
        pltpu.VMEM((PW, EDIM), _f32),
        pltpu.VMEM((CB * HIST, EDIM), _f32),
        pltpu.VMEM((CB * HIST, EDIM), _f32),
        pltpu.VMEM((PW, EDIM), _f32),
        pltpu.SemaphoreType.DMA,
        pltpu.SemaphoreType.DMA,
        pltpu.SemaphoreType.DMA,
        pltpu.SemaphoreType.DMA,
        pltpu.SemaphoreType.DMA,
        pltpu.SemaphoreType.DMA,
    ],
    **_SC_PARAMS,
)(_sc_hist_body)

_sc_flat = functools.partial(
    pl.kernel,
    out_type=[jax.ShapeDtypeStruct((B * EDIM,), _f32),
              jax.ShapeDtypeStruct((B * EDIM,), _f32)],
    scratch_types=[
        pltpu.VMEM((PW,), _i32),
        pltpu.VMEM((PW,), _i32),
        pltpu.VMEM((PW * EDIM,), _i32),
        pltpu.VMEM((PW * EDIM,), _f32),
        pltpu.SemaphoreType.DMA,
        pltpu.SemaphoreType.DMA,
        pltpu.SemaphoreType.DMA,
        pltpu.SemaphoreType.DMA,
    ],
    **_SC_PARAMS,
)(_sc_flat_body)

_sc_cnt = functools.partial(
    pl.kernel,
    out_type=jax.ShapeDtypeStruct((B * NCATP,), _f32),
    scratch_types=[
        pltpu.VMEM((PW * HIST,), _i32),
        pltpu.VMEM((CB * NCATP,), _f32),
        pltpu.VMEM((CB * NCATP,), _f32),
        pltpu.SemaphoreType.DMA,
        pltpu.SemaphoreType.DMA,
    ],
    **_SC_PARAMS,
)(_sc_cnt_body)


_BT = 1024
_BN_SCALE = 1.0 / math.sqrt(1.0 + 1e-3)


def _mlp_body(u, m, c, ms, cnt, ctab, g, bt, W1, b1, a1, W2, b2, a2, W3, b3,
              out):
    cs = jnp.dot(cnt[...], ctab[...], preferred_element_type=_f32)
    x = jnp.concatenate([u[...], m[...], c[...], ms[...], cs], axis=1)
    x = g[...] * (x * _BN_SCALE) + bt[...]
    h = jnp.dot(x, W1[...], preferred_element_type=_f32) + b1[...]
    h = jnp.maximum(h, 0.0) + a1[...] * jnp.minimum(h, 0.0)
    h = jnp.dot(h, W2[...], preferred_element_type=_f32) + b2[...]
    h = jnp.maximum(h, 0.0) + a2[...] * jnp.minimum(h, 0.0)
    z = jnp.dot(h, W3[...], preferred_element_type=_f32) + b3[...]
    z = z - jnp.max(z, axis=1, keepdims=True)
    e = jnp.exp(z)
    out[...] = e / jnp.sum(e, axis=1, keepdims=True) + 1e-8


def _mlp(u, m, c, ms, cnt, ctab, g, bt, W1, b1, a1, W2, b2, a2, W3, b3):
    emb_spec = pl.BlockSpec((_BT, EDIM), lambda i: (i, 0))
    def full(arr):
        return pl.BlockSpec(arr.shape, lambda i: (0,) * arr.ndim)
    return pl.pallas_call(
        _mlp_body,
        grid=(B // _BT,),
        in_specs=[emb_spec] * 4
        + [pl.BlockSpec((_BT, NCATP), lambda i: (i, 0)), full(ctab)]
        + [full(a) for a in (g, bt, W1, b1, a1, W2, b2, a2, W3, b3)],
        out_specs=pl.BlockSpec((_BT, 2), lambda i: (i, 0)),
        out_shape=jax.ShapeDtypeStruct((B, 2), _f32),
    )(u, m, c, ms, cnt, ctab, g, bt, W1, b1, a1, W2, b2, a2, W3, b3)


def kernel(uid_batch_ph, mid_batch_ph, cat_batch_ph, mid_his_batch_ph,
           cat_his_batch_ph, uid_table, mid_table, cat_table, bn_gamma,
           bn_beta, W1, b1, alpha1, W2, b2, alpha2, W3, b3):
    uid = uid_batch_ph.astype(_i32)
    mid = mid_batch_ph.astype(_i32)
    cat = cat_batch_ph.astype(_i32)
    mh = mid_his_batch_ph.astype(_i32).reshape(-1)
    ch = cat_his_batch_ph.astype(_i32).reshape(-1)
    cnt = _sc_cnt(ch).reshape(B, NCATP)
    me, msum = _sc_hist(mid, mh, mid_table)
    uef, cef = _sc_flat(uid, cat, uid_table.T.reshape(-1),
                        cat_table.T.reshape(-1))
    ue = uef.reshape(B, EDIM)
    ce = cef.reshape(B, EDIM)
    ctab = jnp.pad(cat_table, ((0, NCATP - N_CAT), (0, 0)))
    r2 = lambda a: a.reshape(1, -1)
    return _mlp(ue, me, ce, msum, cnt, ctab, r2(bn_gamma), r2(bn_beta),
                W1, r2(b1), r2(alpha1), W2, r2(b2), r2(alpha2),
                W3, r2(b3))


# speedup vs baseline: 1.1935x; 1.1935x over previous
import functools
import math

import jax
import jax.numpy as jnp
from jax import lax
from jax.experimental import pallas as pl
from jax.experimental.pallas import tpu as pltpu
from jax.experimental.pallas import tpu_sc as plsc

B = 4096
HIST = 50
EDIM = 64
N_UID = 100000
N_CAT = 1000
NC, NS = 2, 16
NW = NC * NS
PW = B // NW
CB = 8
NCH = PW // CB
GCH = 80
NG = (CB * HIST) // GCH
_UNR = 5

_f32 = jnp.float32
_i32 = jnp.int32


def _hist_pass(tab, idxv, hb0, hb1, sem0, sem1, sum_all):
    def fire(c, buf, sem):
        for g in range(NG):
            src = tab.at[idxv.at[pl.ds(c * CB * HIST + g * GCH, GCH)]]
            pltpu.async_copy(src, buf.at[pl.ds(g * GCH, GCH)], sem)

    def drain(buf, sem):
        pltpu.make_async_copy(tab.at[pl.ds(0, CB * HIST)], buf, sem).wait()

    def reduce(c, buf):
        for b in range(CB):
            def body(j, acc):
                for u in range(_UNR):
                    r = b * HIST + j * _UNR + u
                    acc = tuple(acc[d] + buf[r, pl.ds(16 * d, 16)]
                                for d in range(4))
                return acc
            z = jnp.zeros((16,), _f32)
            a = lax.fori_loop(0, HIST // _UNR, body, (z, z, z, z))
            row = c * CB + b
            for d in range(4):
                sum_all[row, pl.ds(16 * d, 16)] = a[d]

    fire(0, hb0, sem0)

    def pair(i, carry):
        c0 = 2 * i
        c1 = c0 + 1
        fire(c1, hb1, sem1)
        drain(hb0, sem0)
        reduce(c0, hb0)

        @pl.when(c1 + 1 < NCH)
        def _():
            fire(c1 + 1, hb0, sem0)

        drain(hb1, sem1)
        reduce(c1, hb1)
        return carry

    lax.fori_loop(0, NCH // 2, pair, 0)


def _sc_hist_body(idx_i, hidx_i, tab,
                  emb_o, sum_o,
                  idxv, hidxv, rows, hb0, hb1, ssum,
                  isem, hsem, usem, gsem0, gsem1, osem):
    wid = lax.axis_index("s") * NC + lax.axis_index("c")
    base = wid * PW
    b50 = base * HIST

    d_h = pltpu.async_copy(hidx_i.at[pl.ds(b50, PW * HIST)], hidxv, hsem)
    d_i = pltpu.async_copy(idx_i.at[pl.ds(base, PW)], idxv, isem)
    d_i.wait()
    g = pltpu.async_copy(tab.at[idxv], rows, usem)
    d_h.wait()
    _hist_pass(tab, hidxv, hb0, hb1, gsem0, gsem1, ssum)
    g.wait()
    dst = pl.ds(base, PW)
    o1 = pltpu.async_copy(rows, emb_o.at[dst], osem)
    o2 = pltpu.async_copy(ssum, sum_o.at[dst], osem)
    o1.wait()
    o2.wait()


def _sc_flat_body(uid_i, ut, ue_o, uidv, idxb, rowsf, isem, gsem, osem):
    wid = lax.axis_index("s") * NC + lax.axis_index("c")
    base = wid * PW
    pltpu.async_copy(uid_i.at[pl.ds(base, PW)], uidv, isem).wait()

    foffs = [(lax.iota(_i32, 16) + 16 * k) * N_UID for k in range(4)]

    def build(c, carry):
        for b in range(CB):
            r = c * CB + b
            bc = plsc.load_gather(uidv, [jnp.full((16,), r, _i32)])
            for k in range(4):
                idxb[pl.ds(r * EDIM + 16 * k, 16)] = bc + foffs[k]
        return carry

    lax.fori_loop(0, NCH, build, 0)

    def fire(d, carry):
        src = ut.at[idxb.at[pl.ds(d * 128, 128)]]
        pltpu.async_copy(src, rowsf.at[pl.ds(d * 128, 128)], gsem)
        return carry

    lax.fori_loop(0, PW * EDIM // 128, fire, 0)
    pltpu.make_async_copy(ut.at[pl.ds(0, PW * EDIM)], rowsf, gsem).wait()
    pltpu.async_copy(rowsf, ue_o.at[pl.ds(base * EDIM, PW * EDIM)],
                     osem).wait()


_SC_PARAMS = dict(
    mesh=plsc.VectorSubcoreMesh(core_axis_name="c", subcore_axis_name="s"),
    compiler_params=pltpu.CompilerParams(use_tc_tiling_on_sc=False,
                                         needs_layout_passes=False),
)

_sc_hist = functools.partial(
    pl.kernel,
    out_type=[jax.ShapeDtypeStruct((B, EDIM), _f32),
              jax.ShapeDtypeStruct((B, EDIM), _f32)],
    scratch_types=[
        pltpu.VMEM((PW,), _i32),
        pltpu.VMEM((PW * HIST,), _i32),
        pltpu.VMEM((PW, EDIM), _f32),
        pltpu.VMEM((CB * HIST, EDIM), _f32),
        pltpu.VMEM((CB * HIST, EDIM), _f32),
        pltpu.VMEM((PW, EDIM), _f32),
        pltpu.SemaphoreType.DMA,
        pltpu.SemaphoreType.DMA,
        pltpu.SemaphoreType.DMA,
        pltpu.SemaphoreType.DMA,
        pltpu.SemaphoreType.DMA,
        pltpu.SemaphoreType.DMA,
    ],
    **_SC_PARAMS,
)(_sc_hist_body)

_sc_flat = functools.partial(
    pl.kernel,
    out_type=jax.ShapeDtypeStruct((B * EDIM,), _f32),
    scratch_types=[
        pltpu.VMEM((PW,), _i32),
        pltpu.VMEM((PW * EDIM,), _i32),
        pltpu.VMEM((PW * EDIM,), _f32),
        pltpu.SemaphoreType.DMA,
        pltpu.SemaphoreType.DMA,
        pltpu.SemaphoreType.DMA,
    ],
    **_SC_PARAMS,
)(_sc_flat_body)


_BT = 1024
_BN_SCALE = 1.0 / math.sqrt(1.0 + 1e-3)


def _mlp_body(u, m, c, ms, cs, g, bt, W1, b1, a1, W2, b2, a2, W3, b3, out):
    x = jnp.concatenate([u[...], m[...], c[...], ms[...], cs[...]], axis=1)
    x = g[...] * (x * _BN_SCALE) + bt[...]
    h = jnp.dot(x, W1[...], preferred_element_type=_f32) + b1[...]
    h = jnp.maximum(h, 0.0) + a1[...] * jnp.minimum(h, 0.0)
    h = jnp.dot(h, W2[...], preferred_element_type=_f32) + b2[...]
    h = jnp.maximum(h, 0.0) + a2[...] * jnp.minimum(h, 0.0)
    z = jnp.dot(h, W3[...], preferred_element_type=_f32) + b3[...]
    z = z - jnp.max(z, axis=1, keepdims=True)
    e = jnp.exp(z)
    out[...] = e / jnp.sum(e, axis=1, keepdims=True) + 1e-8


def _mlp(u, m, c, ms, cs, g, bt, W1, b1, a1, W2, b2, a2, W3, b3):
    emb_spec = pl.BlockSpec((_BT, EDIM), lambda i: (i, 0))
    def full(arr):
        return pl.BlockSpec(arr.shape, lambda i: (0,) * arr.ndim)
    return pl.pallas_call(
        _mlp_body,
        grid=(B // _BT,),
        in_specs=[emb_spec] * 5 + [full(a) for a in
                                   (g, bt, W1, b1, a1, W2, b2, a2, W3, b3)],
        out_specs=pl.BlockSpec((_BT, 2), lambda i: (i, 0)),
        out_shape=jax.ShapeDtypeStruct((B, 2), _f32),
    )(u, m, c, ms, cs, g, bt, W1, b1, a1, W2, b2, a2, W3, b3)


def kernel(uid_batch_ph, mid_batch_ph, cat_batch_ph, mid_his_batch_ph,
           cat_his_batch_ph, uid_table, mid_table, cat_table, bn_gamma,
           bn_beta, W1, b1, alpha1, W2, b2, alpha2, W3, b3):
    uid = uid_batch_ph.astype(_i32)
    mid = mid_batch_ph.astype(_i32)
    cat = cat_batch_ph.astype(_i32)
    mh = mid_his_batch_ph.astype(_i32).reshape(-1)
    ch = cat_his_batch_ph.astype(_i32).reshape(-1)
    rep = 4
    cat_rep = jnp.tile(cat_table, (rep, 1))
    ch = ch + (jnp.arange(B * HIST, dtype=_i32) % rep) * N_CAT
    ce, csum = _sc_hist(cat, ch, cat_rep)
    me, msum = _sc_hist(mid, mh, mid_table)
    ue = _sc_flat(uid, uid_table.T.reshape(-1)).reshape(B, EDIM)
    r2 = lambda a: a.reshape(1, -1)
    return _mlp(ue, me, ce, msum, csum, r2(bn_gamma), r2(bn_beta),
                W1, r2(b1), r2(alpha1), W2, r2(b2), r2(alpha2),
                W3, r2(b3))
